# Initial kernel scaffold; baseline (speedup 1.0000x reference)
import jax
import jax.numpy as jnp
from jax.experimental import pallas as pl


def kernel(x, edge_index, edge_weight, W1, b1, W2, b2):
    raise NotImplementedError("write your pallas kernel here")


# speedup vs baseline: 3.4396x; 3.4396x over previous
import jax
import jax.numpy as jnp
from jax import lax
from jax.experimental import pallas as pl
from jax.experimental.pallas import tpu as pltpu
from jax.experimental.pallas import tpu_sc as plsc

_N = 10000
_E = 320000
_D = 128
_NC = 2
_NS = 16
_NW = _NC * _NS
_EPW = _E // _NW
_CH = 80
_NCHUNK = _EPW // _CH
_NP = 10240
_RPT = _NP // _NS
_BN = 1000
_BE = 512


def _node_body(x_ref, w1_ref, b1_ref, xa_ref, bm_ref):
    xb = x_ref[...]
    w1 = w1_ref[...]
    a = lax.dot_general(xb, w1[:, :_D], (((1,), (1,)), ((), ())),
                        preferred_element_type=jnp.float32) + b1_ref[...]
    xa_ref[...] = jnp.concatenate([a, xb], axis=1)
    bm_ref[...] = lax.dot_general(xb, w1[:, _D:], (((1,), (1,)), ((), ())),
                                  preferred_element_type=jnp.float32)


def _gather_body(xa_hbm, bm_hbm, dst_hbm, src_hbm, g1_hbm, g2_hbm,
                 dstv, srcv, bufa, bufb, sema, semb):
    c = lax.axis_index("c")
    s = lax.axis_index("s")
    wid = s * _NC + c
    pltpu.sync_copy(dst_hbm.at[wid], dstv)
    pltpu.sync_copy(src_hbm.at[wid], srcv)
    base = wid * _EPW

    def chunk(j, carry):
        ca = pltpu.async_copy(xa_hbm.at[dstv.at[j]], bufa, sema)
        cb = pltpu.async_copy(bm_hbm.at[srcv.at[j]], bufb, semb)
        ca.wait()
        cb.wait()
        pltpu.sync_copy(bufa, g1_hbm.at[pl.ds(base + j * _CH, _CH)])
        pltpu.sync_copy(bufb, g2_hbm.at[pl.ds(base + j * _CH, _CH)])
        return carry

    lax.fori_loop(0, _NCHUNK, chunk, 0)


def _edge_body(g1_ref, g2_ref, ew_ref, w2_ref, b2_ref, msg_ref):
    g1 = g1_ref[...]
    h = jnp.maximum(g1[:, :_D] + g2_ref[...], 0.0) + g1[:, _D:]
    u = lax.dot_general(h, w2_ref[...], (((1,), (1,)), ((), ())),
                        preferred_element_type=jnp.float32) + b2_ref[...]
    lane4 = lax.broadcasted_iota(jnp.int32, (4, _D), 1)
    row4 = lax.broadcasted_iota(jnp.int32, (4, _D), 0)
    even4 = (lane4 % 2) == 0
    sself = jnp.where((row4 == 0) & even4 | (row4 == 3) & ~even4, 1.0, 0.0)
    sswap = jnp.where((row4 == 2) & even4 | (row4 == 1) & ~even4, 1.0, 0.0)
    ew = ew_ref[...]
    wself = lax.dot_general(ew, sself, (((1,), (0,)), ((), ())),
                            preferred_element_type=jnp.float32)
    wswap = lax.dot_general(ew, sswap, (((1,), (0,)), ((), ())),
                            preferred_element_type=jnp.float32)
    rolled_dn = pltpu.roll(u, _D - 1, 1)
    rolled_up = pltpu.roll(u, 1, 1)
    lane = lax.broadcasted_iota(jnp.int32, u.shape, 1)
    swapped = jnp.where((lane % 2) == 0, rolled_dn, rolled_up)
    msg_ref[...] = u * wself + swapped * wswap


def _scatter_body(msg_hbm, dst_hbm, zer_hbm,
                  pacc_hbm, qacc_hbm,
                  dstj, msgv, hist, acc):
    c = lax.axis_index("c")
    s = lax.axis_index("s")
    wid = s * _NC + c
    pltpu.sync_copy(zer_hbm.at[pl.ds(s * _RPT, _RPT)],
                    acc.at[pl.ds(s * _RPT, _RPT)])

    def zbody(i, carry):
        hist[pl.ds(i * 16, 16)] = jnp.zeros((16,), jnp.float32)
        return carry

    lax.fori_loop(0, _NP // 16, zbody, 0)
    plsc.subcore_barrier()
    base = wid * _EPW
    ones16 = jnp.ones((16,), jnp.float32)
    lanes = jax.lax.iota(jnp.int32, 16)
    masks = [lanes == m for m in range(16)]

    def chunk(j, carry):
        pltpu.sync_copy(dst_hbm.at[wid, j], dstj)
        pltpu.sync_copy(msg_hbm.at[pl.ds(base + j * _CH, _CH)], msgv)
        pltpu.sync_copy(msgv, acc.at[dstj], add=True)
        for k in range(_CH // 16):
            v = dstj[pl.ds(k * 16, 16)]
            for m in range(16):
                plsc.addupdate_scatter(hist, [v], ones16, mask=masks[m])
        return carry

    lax.fori_loop(0, _NCHUNK, chunk, 0)
    plsc.subcore_barrier()
    pltpu.sync_copy(acc.at[pl.ds(s * _RPT, _RPT)],
                    pacc_hbm.at[c, pl.ds(s * _RPT, _RPT)])
    pltpu.sync_copy(hist, qacc_hbm.at[wid])


def _final_body(p_ref, q_ref, o_ref):
    p = p_ref[0] + p_ref[1]
    o_ref[...] = p / jnp.maximum(q_ref[...], 1.0)


def _sc_mesh():
    return plsc.VectorSubcoreMesh(core_axis_name="c", subcore_axis_name="s")


def kernel(x, edge_index, edge_weight, W1, b1, W2, b2):
    dst = edge_index[1].reshape(_NW, _NCHUNK, _CH)
    src = edge_index[0].reshape(_NW, _NCHUNK, _CH)
    ew4 = edge_weight.reshape(_E, 4)
    b1r = b1.reshape(1, _D)
    b2r = b2.reshape(1, _D)
    zer = jnp.zeros((_NP, _D), jnp.float32)

    f32 = jnp.float32
    xa, bm = pl.pallas_call(
        _node_body,
        grid=(_N // _BN,),
        in_specs=[
            pl.BlockSpec((_BN, _D), lambda i: (i, 0)),
            pl.BlockSpec((_D, 2 * _D), lambda i: (0, 0)),
            pl.BlockSpec((1, _D), lambda i: (0, 0)),
        ],
        out_specs=[
            pl.BlockSpec((_BN, 2 * _D), lambda i: (i, 0)),
            pl.BlockSpec((_BN, _D), lambda i: (i, 0)),
        ],
        out_shape=[
            jax.ShapeDtypeStruct((_N, 2 * _D), f32),
            jax.ShapeDtypeStruct((_N, _D), f32),
        ],
    )(x, W1, b1r)

    g1, g2 = pl.kernel(
        _gather_body,
        out_type=[
            jax.ShapeDtypeStruct((_E, 2 * _D), f32),
            jax.ShapeDtypeStruct((_E, _D), f32),
        ],
        mesh=_sc_mesh(),
        scratch_types=[
            pltpu.VMEM((_NCHUNK, _CH), jnp.int32),
            pltpu.VMEM((_NCHUNK, _CH), jnp.int32),
            pltpu.VMEM((_CH, 2 * _D), f32),
            pltpu.VMEM((_CH, _D), f32),
            pltpu.SemaphoreType.DMA,
            pltpu.SemaphoreType.DMA,
        ],
    )(xa, bm, dst, src)

    msg = pl.pallas_call(
        _edge_body,
        grid=(_E // _BE,),
        in_specs=[
            pl.BlockSpec((_BE, 2 * _D), lambda i: (i, 0)),
            pl.BlockSpec((_BE, _D), lambda i: (i, 0)),
            pl.BlockSpec((_BE, 4), lambda i: (i, 0)),
            pl.BlockSpec((_D, _D), lambda i: (0, 0)),
            pl.BlockSpec((1, _D), lambda i: (0, 0)),
        ],
        out_specs=pl.BlockSpec((_BE, _D), lambda i: (i, 0)),
        out_shape=jax.ShapeDtypeStruct((_E, _D), f32),
    )(g1, g2, ew4, W2, b2r)

    pacc, qacc = pl.kernel(
        _scatter_body,
        out_type=[
            jax.ShapeDtypeStruct((_NC, _NP, _D), f32),
            jax.ShapeDtypeStruct((_NW, _NP), f32),
        ],
        mesh=_sc_mesh(),
        scratch_types=[
            pltpu.VMEM((_CH,), jnp.int32),
            pltpu.VMEM((_CH, _D), f32),
            pltpu.VMEM((_NP,), f32),
            pltpu.VMEM_SHARED((_NP, _D), f32),
        ],
        compiler_params=pltpu.CompilerParams(needs_layout_passes=False),
    )(msg, dst, zer)
    cnt = jnp.sum(qacc, axis=0).reshape(_NP, 1)

    out = pl.pallas_call(
        _final_body,
        grid=(_N // _BN,),
        in_specs=[
            pl.BlockSpec((_NC, _BN, _D), lambda i: (0, i, 0)),
            pl.BlockSpec((_BN, 1), lambda i: (i, 0)),
        ],
        out_specs=pl.BlockSpec((_BN, _D), lambda i: (i, 0)),
        out_shape=jax.ShapeDtypeStruct((_N, _D), f32),
    )(pacc, cnt)
    return out


# speedup vs baseline: 3.4986x; 1.0171x over previous
import jax
import jax.numpy as jnp
from jax import lax
from jax.experimental import pallas as pl
from jax.experimental.pallas import tpu as pltpu
from jax.experimental.pallas import tpu_sc as plsc

_N = 10000
_E = 320000
_D = 128
_NC = 2
_NS = 16
_NW = _NC * _NS
_EPW = _E // _NW
_CH = 80
_NCHUNK = _EPW // _CH
_NP = 10240
_RPT = _NP // _NS
_BN = 1000
_BE = 512


def _node_body(x_ref, w1_ref, b1_ref, xa_ref, bm_ref):
    xb = x_ref[...]
    w1 = w1_ref[...]
    a = lax.dot_general(xb, w1[:, :_D], (((1,), (1,)), ((), ())),
                        preferred_element_type=jnp.float32) + b1_ref[...]
    xa_ref[...] = jnp.concatenate([a, xb], axis=1)
    bm_ref[...] = lax.dot_general(xb, w1[:, _D:], (((1,), (1,)), ((), ())),
                                  preferred_element_type=jnp.float32)


def _gather_body(xa_hbm, bm_hbm, dst_hbm, src_hbm, g_hbm,
                 dstv, srcv, ba0, ba1, bb0, bb1, hbuf,
                 sa0, sa1, sb0, sb1):
    c = lax.axis_index("c")
    s = lax.axis_index("s")
    wid = s * _NC + c
    pltpu.sync_copy(dst_hbm.at[wid], dstv)
    pltpu.sync_copy(src_hbm.at[wid], srcv)
    base = wid * _EPW
    ba = (ba0, ba1)
    bb = (bb0, bb1)
    sa = (sa0, sa1)
    sb = (sb0, sb1)

    def start(j, b):
        pltpu.async_copy(xa_hbm.at[dstv.at[j]], ba[b], sa[b])
        pltpu.async_copy(bm_hbm.at[srcv.at[j]], bb[b], sb[b])

    def wait(j, b):
        pltpu.make_async_copy(xa_hbm.at[dstv.at[j]], ba[b], sa[b]).wait()
        pltpu.make_async_copy(bm_hbm.at[srcv.at[j]], bb[b], sb[b]).wait()

    def compute(j, b):
        av = ba[b]
        bv = bb[b]

        def erow(e, carry):
            for k in range(_D // 16):
                a = av[e, pl.ds(k * 16, 16)]
                g = bv[e, pl.ds(k * 16, 16)]
                xi = av[e, pl.ds(_D + k * 16, 16)]
                hbuf[e, pl.ds(k * 16, 16)] = jnp.maximum(a + g, 0.0) + xi
            return carry

        lax.fori_loop(0, _CH, erow, 0)
        pltpu.sync_copy(hbuf, g_hbm.at[pl.ds(base + j * _CH, _CH)])

    start(0, 0)

    def outer(i, carry):
        for b in (0, 1):
            j = 2 * i + b
            wait(j, b)

            @pl.when(j + 1 < _NCHUNK)
            def _():
                start(j + 1, 1 - b)

            compute(j, b)
        return carry

    lax.fori_loop(0, (_NCHUNK - 1) // 2, outer, 0)
    wait(_NCHUNK - 1, 0)
    compute(_NCHUNK - 1, 0)


def _edge_body(g_ref, ew_ref, w2_ref, b2_ref, msg_ref):
    u = lax.dot_general(g_ref[...], w2_ref[...], (((1,), (1,)), ((), ())),
                        preferred_element_type=jnp.float32) + b2_ref[...]
    lane4 = lax.broadcasted_iota(jnp.int32, (4, _D), 1)
    row4 = lax.broadcasted_iota(jnp.int32, (4, _D), 0)
    even4 = (lane4 % 2) == 0
    sself = jnp.where((row4 == 0) & even4 | (row4 == 3) & ~even4, 1.0, 0.0)
    sswap = jnp.where((row4 == 2) & even4 | (row4 == 1) & ~even4, 1.0, 0.0)
    ew = ew_ref[...]
    wself = lax.dot_general(ew, sself, (((1,), (0,)), ((), ())),
                            preferred_element_type=jnp.float32)
    wswap = lax.dot_general(ew, sswap, (((1,), (0,)), ((), ())),
                            preferred_element_type=jnp.float32)
    rolled_dn = pltpu.roll(u, _D - 1, 1)
    rolled_up = pltpu.roll(u, 1, 1)
    lane = lax.broadcasted_iota(jnp.int32, u.shape, 1)
    swapped = jnp.where((lane % 2) == 0, rolled_dn, rolled_up)
    msg_ref[...] = u * wself + swapped * wswap


def _scatter_body(msg_hbm, dstf_hbm, zer_hbm, pacc_hbm, qacc_hbm,
                  dj0, dj1, mv0, mv1, hist, acc,
                  sd0, sd1, sm0, sm1):
    c = lax.axis_index("c")
    s = lax.axis_index("s")
    wid = s * _NC + c
    pltpu.sync_copy(zer_hbm.at[pl.ds(s * _RPT, _RPT)],
                    acc.at[pl.ds(s * _RPT, _RPT)])

    def zbody(i, carry):
        hist[pl.ds(i * 16, 16)] = jnp.zeros((16,), jnp.float32)
        return carry

    lax.fori_loop(0, _NP // 16, zbody, 0)
    plsc.subcore_barrier()
    base = wid * _EPW
    ones16 = jnp.ones((16,), jnp.float32)
    lanes = lax.iota(jnp.int32, 16)
    masks = [lanes == m for m in range(16)]
    dj = (dj0, dj1)
    mv = (mv0, mv1)
    sd = (sd0, sd1)
    sm = (sm0, sm1)

    def start(j, b):
        pltpu.async_copy(dstf_hbm.at[pl.ds(base + j * _CH, _CH)], dj[b], sd[b])
        pltpu.async_copy(msg_hbm.at[pl.ds(base + j * _CH, _CH)], mv[b], sm[b])

    def wait(j, b):
        pltpu.make_async_copy(dstf_hbm.at[pl.ds(base + j * _CH, _CH)],
                              dj[b], sd[b]).wait()
        pltpu.make_async_copy(msg_hbm.at[pl.ds(base + j * _CH, _CH)],
                              mv[b], sm[b]).wait()

    def process(j, b):
        pltpu.sync_copy(mv[b], acc.at[dj[b]], add=True)
        for k in range(_CH // 16):
            v = dj[b][pl.ds(k * 16, 16)]
            for m in range(16):
                plsc.addupdate_scatter(hist, [v], ones16, mask=masks[m])

    start(0, 0)

    def outer(i, carry):
        for b in (0, 1):
            j = 2 * i + b
            wait(j, b)

            @pl.when(j + 1 < _NCHUNK)
            def _():
                start(j + 1, 1 - b)

            process(j, b)
        return carry

    lax.fori_loop(0, (_NCHUNK - 1) // 2, outer, 0)
    wait(_NCHUNK - 1, 0)
    process(_NCHUNK - 1, 0)
    plsc.subcore_barrier()
    pltpu.sync_copy(acc.at[pl.ds(s * _RPT, _RPT)],
                    pacc_hbm.at[c, pl.ds(s * _RPT, _RPT)])
    pltpu.sync_copy(hist, qacc_hbm.at[wid])


def _final_body(p_ref, q_ref, o_ref):
    p = p_ref[0] + p_ref[1]
    o_ref[...] = p / jnp.maximum(q_ref[...], 1.0)


def _sc_mesh():
    return plsc.VectorSubcoreMesh(core_axis_name="c", subcore_axis_name="s")


def kernel(x, edge_index, edge_weight, W1, b1, W2, b2):
    dst = edge_index[1].reshape(_NW, _NCHUNK, _CH)
    src = edge_index[0].reshape(_NW, _NCHUNK, _CH)
    ew4 = edge_weight.reshape(_E, 4)
    b1r = b1.reshape(1, _D)
    b2r = b2.reshape(1, _D)
    zer = jnp.zeros((_NP, _D), jnp.float32)

    f32 = jnp.float32
    xa, bm = pl.pallas_call(
        _node_body,
        grid=(_N // _BN,),
        in_specs=[
            pl.BlockSpec((_BN, _D), lambda i: (i, 0)),
            pl.BlockSpec((_D, 2 * _D), lambda i: (0, 0)),
            pl.BlockSpec((1, _D), lambda i: (0, 0)),
        ],
        out_specs=[
            pl.BlockSpec((_BN, 2 * _D), lambda i: (i, 0)),
            pl.BlockSpec((_BN, _D), lambda i: (i, 0)),
        ],
        out_shape=[
            jax.ShapeDtypeStruct((_N, 2 * _D), f32),
            jax.ShapeDtypeStruct((_N, _D), f32),
        ],
    )(x, W1, b1r)

    g = pl.kernel(
        _gather_body,
        out_type=jax.ShapeDtypeStruct((_E, _D), f32),
        mesh=_sc_mesh(),
        scratch_types=[
            pltpu.VMEM((_NCHUNK, _CH), jnp.int32),
            pltpu.VMEM((_NCHUNK, _CH), jnp.int32),
            pltpu.VMEM((_CH, 2 * _D), f32),
            pltpu.VMEM((_CH, 2 * _D), f32),
            pltpu.VMEM((_CH, _D), f32),
            pltpu.VMEM((_CH, _D), f32),
            pltpu.VMEM((_CH, _D), f32),
            pltpu.SemaphoreType.DMA,
            pltpu.SemaphoreType.DMA,
            pltpu.SemaphoreType.DMA,
            pltpu.SemaphoreType.DMA,
        ],
    )(xa, bm, dst, src)

    msg = pl.pallas_call(
        _edge_body,
        grid=(_E // _BE,),
        in_specs=[
            pl.BlockSpec((_BE, _D), lambda i: (i, 0)),
            pl.BlockSpec((_BE, 4), lambda i: (i, 0)),
            pl.BlockSpec((_D, _D), lambda i: (0, 0)),
            pl.BlockSpec((1, _D), lambda i: (0, 0)),
        ],
        out_specs=pl.BlockSpec((_BE, _D), lambda i: (i, 0)),
        out_shape=jax.ShapeDtypeStruct((_E, _D), f32),
    )(g, ew4, W2, b2r)

    pacc, qacc = pl.kernel(
        _scatter_body,
        out_type=[
            jax.ShapeDtypeStruct((_NC, _NP, _D), f32),
            jax.ShapeDtypeStruct((_NW, _NP), f32),
        ],
        mesh=_sc_mesh(),
        scratch_types=[
            pltpu.VMEM((_CH,), jnp.int32),
            pltpu.VMEM((_CH,), jnp.int32),
            pltpu.VMEM((_CH, _D), f32),
            pltpu.VMEM((_CH, _D), f32),
            pltpu.VMEM((_NP,), f32),
            pltpu.VMEM_SHARED((_NP, _D), f32),
            pltpu.SemaphoreType.DMA,
            pltpu.SemaphoreType.DMA,
            pltpu.SemaphoreType.DMA,
            pltpu.SemaphoreType.DMA,
        ],
        compiler_params=pltpu.CompilerParams(needs_layout_passes=False),
    )(msg, edge_index[1], zer)
    cnt = jnp.sum(qacc, axis=0).reshape(_NP, 1)

    out = pl.pallas_call(
        _final_body,
        grid=(_N // _BN,),
        in_specs=[
            pl.BlockSpec((_NC, _BN, _D), lambda i: (0, i, 0)),
            pl.BlockSpec((_BN, 1), lambda i: (i, 0)),
        ],
        out_specs=pl.BlockSpec((_BN, _D), lambda i: (i, 0)),
        out_shape=jax.ShapeDtypeStruct((_N, _D), f32),
    )(pacc, cnt)
    return out
